# baseline (speedup 1.0000x reference)
import functools

import jax
import jax.numpy as jnp
from jax import lax
from jax.experimental import pallas as pl
from jax.experimental.pallas import tpu as pltpu
from jax.experimental.pallas import tpu_sc as plsc

N_NODES = 10000
N_PAD = 10240
E_EDGES = 320000
WORD = 300
NC, NS = 2, 16
EB = 128
NB = 80
GRP = 2
STRIPE = N_PAD // NS
EB2 = 64
NBLK_P = 81
NPASS = 2
NBLK = NBLK_P * NPASS
EPT_P = EB2 * NBLK
EPP = EB2 * NBLK_P
D = 128
BLK = 2000


def _mesh():
    return plsc.VectorSubcoreMesh(
        core_axis_name="c", subcore_axis_name="s", num_cores=NC, num_subcores=NS
    )


def _sc_deg_body(dst_hbm, ew_hbm, out_hbm, dst_v, ew_v, zbuf, acc_sh):
    c = lax.axis_index("c")
    s = lax.axis_index("s")
    pltpu.sync_copy(dst_hbm.at[c, s], dst_v)
    pltpu.sync_copy(ew_hbm.at[c, s], ew_v)

    def zf(i, carry):
        zbuf[pl.ds(i * 16, 16)] = jnp.zeros((16,), jnp.float32)
        return carry

    lax.fori_loop(0, STRIPE // 16, zf, None)
    pltpu.sync_copy(zbuf, acc_sh.at[pl.ds(s * STRIPE, STRIPE)])
    plsc.subcore_barrier()

    def eb_fn(j, carry):
        pltpu.sync_copy(ew_v.at[pl.ds(j * EB, EB)], acc_sh.at[dst_v.at[j]], add=True)
        return carry

    lax.fori_loop(0, NB, eb_fn, None)
    plsc.subcore_barrier()
    pltpu.sync_copy(
        acc_sh.at[pl.ds(s * STRIPE, STRIPE)], out_hbm.at[c, pl.ds(s * STRIPE, STRIPE)]
    )


def _sc_deg(dst_p, ew_p):
    fn = pl.kernel(
        _sc_deg_body,
        out_type=jax.ShapeDtypeStruct((NC, N_PAD), jnp.float32),
        mesh=_mesh(),
        scratch_types=[
            pltpu.VMEM((NB, EB), jnp.int32),
            pltpu.VMEM((NB * EB,), jnp.float32),
            pltpu.VMEM((STRIPE,), jnp.float32),
            pltpu.VMEM_SHARED((N_PAD,), jnp.float32),
        ],
    )
    return fn(dst_p, ew_p)


def _sc_edges(hw, packed_p, ew_p):
    def body(hw_hbm, pk_hbm, ew_hbm, out_hbm,
             pk_v, ew_v, rows, idx, acc_sh,
             gsem0, gsem1, gsem2, ssem0, ssem1, ssem2):
        gsems = [gsem0, gsem1, gsem2]
        ssems = [ssem0, ssem1, ssem2]
        c = lax.axis_index("c")
        s = lax.axis_index("s")

        def zf(r, carry):
            for cc in range(D // 16):
                rows[0, r, pl.ds(cc * 16, 16)] = jnp.zeros((16,), jnp.float32)
            return carry

        lax.fori_loop(0, EB2, zf, None)
        base = pl.multiple_of(s * STRIPE, 8)
        for t in range(STRIPE // EB2):
            pltpu.async_copy(
                rows.at[0], acc_sh.at[pl.ds(base + t * EB2, EB2)], ssem0
            )
        for t in range(STRIPE // EB2):
            pltpu.make_async_copy(
                rows.at[0], acc_sh.at[pl.ds(0, EB2)], ssem0
            ).wait()
        plsc.subcore_barrier()

        def unpack(j, b):
            def up(k, carry):
                pk16 = pk_v[j, pl.ds(k * 16, 16)]
                idx[b, 0, pl.ds(k * 16, 16)] = pk16 & 0xFFFF
                idx[b, 1, pl.ds(k * 16, 16)] = pk16 >> 16
                return carry

            lax.fori_loop(0, EB2 // 16, up, None)

        def pass_fn(p, carry):
            pltpu.sync_copy(pk_hbm.at[c, s, p], pk_v)
            pltpu.sync_copy(ew_hbm.at[c, s, p], ew_v)
            for b in range(2):
                unpack(jnp.int32(b), b)
                pltpu.async_copy(hw_hbm.at[idx.at[b, 0]], rows.at[b], gsems[b])

            def step(t, carry2):
                for k in range(3):
                    j = 3 * t + k
                    b = k
                    b3 = (k + 2) % 3

                    @pl.when(j >= 1)
                    def _wait_sc():
                        pltpu.make_async_copy(
                            rows.at[b3], acc_sh.at[pl.ds(0, EB2)], ssems[b3]
                        ).wait()

                    @pl.when(j < NBLK_P - 2)
                    def _prefetch():
                        unpack(j + 2, b3)
                        pltpu.async_copy(
                            hw_hbm.at[idx.at[b3, 0]], rows.at[b3], gsems[b3]
                        )

                    pltpu.make_async_copy(
                        hw_hbm.at[pl.ds(0, EB2)], rows.at[b], gsems[b]
                    ).wait()

                    def scale(gi, carry3, _b=b):
                        wv = ew_v[j, pl.ds(gi * 16, 16)]
                        for i in range(16):
                            w = jnp.full((16,), wv[i], jnp.float32)
                            r = gi * 16 + i
                            for cc in range(D // 16):
                                rows[_b, r, pl.ds(cc * 16, 16)] = (
                                    rows[_b, r, pl.ds(cc * 16, 16)] * w
                                )
                        return carry3

                    lax.fori_loop(0, EB2 // 16, scale, None)
                    pltpu.async_copy(
                        rows.at[b], acc_sh.at[idx.at[b, 1]], ssems[b], add=True
                    )
                return carry2

            lax.fori_loop(0, NBLK_P // 3, step, None)
            pltpu.make_async_copy(
                rows.at[(NBLK_P - 1) % 3],
                acc_sh.at[pl.ds(0, EB2)],
                ssems[(NBLK_P - 1) % 3],
            ).wait()
            return carry

        lax.fori_loop(0, NPASS, pass_fn, None)
        plsc.subcore_barrier()
        pltpu.sync_copy(
            acc_sh.at[pl.ds(base, STRIPE)],
            out_hbm.at[c, pl.ds(base, STRIPE)],
        )

    fn = pl.kernel(
        body,
        out_type=jax.ShapeDtypeStruct((NC, N_PAD, D), jnp.float32),
        mesh=_mesh(),
        scratch_types=[
            pltpu.VMEM((NBLK_P, EB2), jnp.int32),
            pltpu.VMEM((NBLK_P, EB2), jnp.float32),
            pltpu.VMEM((3, EB2, D), jnp.float32),
            pltpu.VMEM((3, 2, EB2), jnp.int32),
            pltpu.VMEM_SHARED((N_PAD, D), jnp.float32),
            pltpu.SemaphoreType.DMA,
            pltpu.SemaphoreType.DMA,
            pltpu.SemaphoreType.DMA,
            pltpu.SemaphoreType.DMA,
            pltpu.SemaphoreType.DMA,
            pltpu.SemaphoreType.DMA,
        ],
    )
    return fn(hw, packed_p, ew_p)


def _tc_front_body(x_ref, degp_ref, Ww_ref, bw_ref, Wr_ref, br_ref, W1_ref, hw_ref):
    xb = x_ref[...]
    w = jnp.dot(xb[:, :WORD], Ww_ref[...], preferred_element_type=jnp.float32)
    r = jnp.dot(xb[:, WORD:], Wr_ref[...], preferred_element_type=jnp.float32)
    h0 = jnp.maximum(
        jnp.concatenate([w + bw_ref[...], r + br_ref[...]], axis=1), 0.0
    )
    dinv = lax.rsqrt(degp_ref[:, 0] + degp_ref[:, 1] + 1.0)
    hw_ref[...] = (
        jnp.dot(h0, W1_ref[...], preferred_element_type=jnp.float32) * dinv[:, None]
    )


def _tc_mid_body(p_ref, hw1_ref, degp_ref, attr_ref, W2a_ref, W2b_ref, b1_ref,
                 hw2_ref, asum_ref):
    dinv = lax.rsqrt(degp_ref[:, 0] + degp_ref[:, 1] + 1.0)[:, None]
    tot = p_ref[0] + p_ref[1] + hw1_ref[...]
    h1 = jnp.maximum(tot * dinv + b1_ref[...], 0.0)
    ab = attr_ref[...]
    hw2_ref[...] = (
        jnp.dot(h1, W2a_ref[...], preferred_element_type=jnp.float32)
        + jnp.dot(ab, W2b_ref[...], preferred_element_type=jnp.float32)
    ) * dinv
    asum_ref[...] = jnp.sum(ab, axis=0).reshape(1, 1, 16)


def _tc_back_body(p_ref, hw2_ref, degp_ref, b2_ref, hsum_ref):
    dinv = lax.rsqrt(degp_ref[:, 0] + degp_ref[:, 1] + 1.0)[:, None]
    h2 = jnp.maximum((p_ref[0] + p_ref[1] + hw2_ref[...]) * dinv + b2_ref[...], 0.0)
    hsum_ref[...] = jnp.sum(h2, axis=0).reshape(1, 1, D)


def _tc_final_body(hs_ref, as_ref, Wma_ref, Wmb_ref, bm_ref, out_ref):
    ph = jnp.sum(hs_ref[...], axis=(0, 1)).reshape(1, D) * (1.0 / N_NODES)
    pa = jnp.sum(as_ref[...], axis=(0, 1)).reshape(1, 16) * (1.0 / N_NODES)
    o = (
        jnp.dot(ph, Wma_ref[...], preferred_element_type=jnp.float32)
        + jnp.dot(pa, Wmb_ref[...], preferred_element_type=jnp.float32)
        + bm_ref[...]
    )
    out_ref[...] = jnp.maximum(o, 0.0)


def _full(shape):
    return pl.BlockSpec(shape, lambda i: tuple(0 for _ in shape))


def kernel(x, attributes, edge_index, edge_weight, W_word, b_word, W_rgb, b_rgb,
           W1, b1, W2, b2, W_map, b_map):
    grid = N_NODES // BLK
    f32 = jnp.float32

    pad = NC * NS * NB * EB - E_EDGES
    dst_p = jnp.pad(edge_index[1], (0, pad)).reshape(NC, NS, NB, EB)
    ew_p = jnp.pad(edge_weight, (0, pad)).reshape(NC, NS, NB * EB)
    pad2 = NC * NS * EPT_P - E_EDGES
    packed = edge_index[0] | (edge_index[1] << 16)
    packed_p = jnp.pad(packed, (0, pad2)).reshape(NC, NS, NPASS, NBLK_P, EB2)
    ew_p2 = jnp.pad(edge_weight, (0, pad2)).reshape(NC, NS, NPASS, NBLK_P, EB2)
    bw2, br2 = b_word.reshape(1, 64), b_rgb.reshape(1, 64)
    b1r, b2r, bmr = b1.reshape(1, D), b2.reshape(1, D), b_map.reshape(1, D)
    W2a, W2b = W2[:D], W2[D:]
    Wma, Wmb = W_map[:D], W_map[D:]

    degp = _sc_deg(dst_p, ew_p).T

    hw1 = pl.pallas_call(
        _tc_front_body,
        grid=(grid,),
        in_specs=[
            pl.BlockSpec((BLK, 812), lambda i: (i, 0)),
            pl.BlockSpec((BLK, 2), lambda i: (i, 0)),
            _full((WORD, 64)),
            _full((1, 64)),
            _full((512, 64)),
            _full((1, 64)),
            _full((D, D)),
        ],
        out_specs=pl.BlockSpec((BLK, D), lambda i: (i, 0)),
        out_shape=jax.ShapeDtypeStruct((N_NODES, D), f32),
    )(x, degp, W_word, bw2, W_rgb, br2, W1)

    p1 = _sc_edges(hw1, packed_p, ew_p2)

    hw2, asum = pl.pallas_call(
        _tc_mid_body,
        grid=(grid,),
        in_specs=[
            pl.BlockSpec((2, BLK, D), lambda i: (0, i, 0)),
            pl.BlockSpec((BLK, D), lambda i: (i, 0)),
            pl.BlockSpec((BLK, 2), lambda i: (i, 0)),
            pl.BlockSpec((BLK, 16), lambda i: (i, 0)),
            _full((D, D)),
            _full((16, D)),
            _full((1, D)),
        ],
        out_specs=[
            pl.BlockSpec((BLK, D), lambda i: (i, 0)),
            pl.BlockSpec((1, 1, 16), lambda i: (i, 0, 0)),
        ],
        out_shape=[
            jax.ShapeDtypeStruct((N_NODES, D), f32),
            jax.ShapeDtypeStruct((grid, 1, 16), f32),
        ],
    )(p1, hw1, degp, attributes, W2a, W2b, b1r)

    p2 = _sc_edges(hw2, packed_p, ew_p2)

    hsum = pl.pallas_call(
        _tc_back_body,
        grid=(grid,),
        in_specs=[
            pl.BlockSpec((2, BLK, D), lambda i: (0, i, 0)),
            pl.BlockSpec((BLK, D), lambda i: (i, 0)),
            pl.BlockSpec((BLK, 2), lambda i: (i, 0)),
            _full((1, D)),
        ],
        out_specs=pl.BlockSpec((1, 1, D), lambda i: (i, 0, 0)),
        out_shape=jax.ShapeDtypeStruct((grid, 1, D), f32),
    )(p2, hw2, degp, b2r)

    out = pl.pallas_call(
        _tc_final_body,
        grid=(1,),
        in_specs=[
            _full((grid, 1, D)),
            _full((grid, 1, 16)),
            _full((D, D)),
            _full((16, D)),
            _full((1, D)),
        ],
        out_specs=_full((1, D)),
        out_shape=jax.ShapeDtypeStruct((1, D), f32),
    )(hsum, asum, Wma, Wmb, bmr)
    return out
